# Initial kernel scaffold; baseline (speedup 1.0000x reference)
import jax
import jax.numpy as jnp
from jax.experimental import pallas as pl


def kernel(wq, bq, wk, bk, wv, bv, wo, bo, off_dw_w, off_dw_b, off_ln_g, off_ln_b, off_pw_w, rpe_w, rpe_b, x):
    raise NotImplementedError("write your pallas kernel here")


# speedup vs baseline: 1.0000x; 1.0000x over previous
import functools

import jax
import jax.numpy as jnp
from jax import lax
from jax.experimental import pallas as pl
from jax.experimental.pallas import tpu as pltpu


def _choose_tile(S, max_tile=512):
    if S % 128 != 0:
        return S
    t = min(max_tile, S)
    t -= t % 128
    while t > 0:
        if S % t == 0:
            return t
        t -= 128
    return S


def _conv1x1_kernel(x_ref, w_ref, b_ref, o_ref):
    x = x_ref[0].astype(jnp.float32)
    w = w_ref[...].astype(jnp.float32)
    y = jnp.dot(w, x, preferred_element_type=jnp.float32)
    y = y + b_ref[...].astype(jnp.float32)
    o_ref[0] = y.astype(o_ref.dtype)


def _conv1x1_res_kernel(x_ref, r_ref, w_ref, b_ref, o_ref):
    x = x_ref[0].astype(jnp.float32) + r_ref[0].astype(jnp.float32)
    w = w_ref[...].astype(jnp.float32)
    y = jnp.dot(w, x, preferred_element_type=jnp.float32)
    y = y + b_ref[...].astype(jnp.float32)
    o_ref[0] = y.astype(o_ref.dtype)


def conv1x1_pallas(x, w, b, residual=None, *, max_tile=512):
    B, C_in, S = x.shape
    C_out = w.shape[0]
    tS = _choose_tile(S, max_tile)
    grid = (B, S // tS)

    x_spec = pl.BlockSpec((1, C_in, tS), lambda bi, si: (bi, 0, si))
    w_spec = pl.BlockSpec((C_out, C_in), lambda bi, si: (0, 0))
    b_spec = pl.BlockSpec((C_out, 1), lambda bi, si: (0, 0))
    o_spec = pl.BlockSpec((1, C_out, tS), lambda bi, si: (bi, 0, si))
    b2 = b.reshape(C_out, 1)

    if residual is None:
        kern = _conv1x1_kernel
        operands = (x, w, b2)
        in_specs = [x_spec, w_spec, b_spec]
    else:
        kern = _conv1x1_res_kernel
        operands = (x, residual, w, b2)
        in_specs = [x_spec, x_spec, w_spec, b_spec]

    return pl.pallas_call(
        kern,
        out_shape=jax.ShapeDtypeStruct((B, C_out, S), x.dtype),
        grid_spec=pltpu.PrefetchScalarGridSpec(
            num_scalar_prefetch=0,
            grid=grid,
            in_specs=in_specs,
            out_specs=o_spec,
        ),
        compiler_params=pltpu.CompilerParams(
            dimension_semantics=("parallel", "parallel")),
    )(*operands)


def _na_kernel(q_ref, k_ref, v_ref, o_ref, *, scale, kk2):
    q = q_ref[0].astype(jnp.float32)

    logits = []
    for r in range(kk2):
        kr = k_ref[0, r].astype(jnp.float32)
        logits.append(jnp.sum(q * kr, axis=0, keepdims=True) * scale)
    s = jnp.concatenate(logits, axis=0)

    m = jnp.max(s, axis=0, keepdims=True)
    p = jnp.exp(s - m)
    denom = jnp.sum(p, axis=0, keepdims=True)
    inv = pl.reciprocal(denom, approx=False)

    acc = jnp.zeros_like(q)
    for r in range(kk2):
        vr = v_ref[0, r].astype(jnp.float32)
        acc = acc + p[r:r + 1, :] * vr
    o_ref[0] = (acc * inv).astype(o_ref.dtype)


def na2d_pallas(q, k_nbr, v_nbr, *, scale, max_tile=512):
    BG, c, S = q.shape
    KK = k_nbr.shape[1]
    tS = _choose_tile(S, max_tile)
    grid = (BG, S // tS)
    kern = functools.partial(_na_kernel, scale=scale, kk2=KK)

    return pl.pallas_call(
        kern,
        out_shape=jax.ShapeDtypeStruct((BG, c, S), q.dtype),
        grid_spec=pltpu.PrefetchScalarGridSpec(
            num_scalar_prefetch=0,
            grid=grid,
            in_specs=[
                pl.BlockSpec((1, c, tS), lambda bi, si: (bi, 0, si)),
                pl.BlockSpec((1, KK, c, tS), lambda bi, si: (bi, 0, 0, si)),
                pl.BlockSpec((1, KK, c, tS), lambda bi, si: (bi, 0, 0, si)),
            ],
            out_specs=pl.BlockSpec((1, c, tS), lambda bi, si: (bi, 0, si)),
        ),
        compiler_params=pltpu.CompilerParams(
            dimension_semantics=("parallel", "parallel")),
    )(q, k_nbr, v_nbr)


def _depthwise_conv(x, w, b, *, stride=1, padding=0):
    C = x.shape[1]
    y = lax.conv_general_dilated(
        x, w, window_strides=(stride, stride),
        padding=[(padding, padding), (padding, padding)],
        dimension_numbers=("NCHW", "OIHW", "NCHW"),
        feature_group_count=C)
    if b is not None:
        y = y + b[None, :, None, None]
    return y


def _layernorm2d(x, gamma, beta, eps=1e-6):
    u = jnp.mean(x, axis=1, keepdims=True)
    s = jnp.mean((x - u) ** 2, axis=1, keepdims=True)
    xn = (x - u) / jnp.sqrt(s + eps)
    return gamma[None, :, None, None] * xn + beta[None, :, None, None]


def _ref_points(Hk, Wk, BG, dtype):
    ref_y = (jnp.arange(Hk, dtype=dtype) + 0.5) / (Hk - 1.0) * 2.0 - 1.0
    ref_x = (jnp.arange(Wk, dtype=dtype) + 0.5) / (Wk - 1.0) * 2.0 - 1.0
    ry, rx = jnp.meshgrid(ref_y, ref_x, indexing="ij")
    ref = jnp.stack([ry, rx], axis=-1)
    return jnp.broadcast_to(ref[None], (BG, Hk, Wk, 2))


def _grid_sample_bilinear(img, grid):
    N, C, H, W = img.shape
    gx = (grid[..., 0] + 1.0) * 0.5 * (W - 1)
    gy = (grid[..., 1] + 1.0) * 0.5 * (H - 1)
    x0 = jnp.floor(gx)
    y0 = jnp.floor(gy)
    x1 = x0 + 1.0
    y1 = y0 + 1.0
    wx1 = gx - x0
    wx0 = 1.0 - wx1
    wy1 = gy - y0
    wy0 = 1.0 - wy1
    flat = img.reshape(N, C, H * W)

    def gather(yi, xi):
        valid = (xi >= 0) & (xi <= W - 1) & (yi >= 0) & (yi <= H - 1)
        xc = jnp.clip(xi, 0, W - 1).astype(jnp.int32)
        yc = jnp.clip(yi, 0, H - 1).astype(jnp.int32)
        idx = (yc * W + xc).reshape(N, 1, -1)
        g = jnp.take_along_axis(flat, idx, axis=2).reshape(N, C, *xi.shape[1:])
        return g * valid.astype(img.dtype)[:, None]

    out = (gather(y0, x0) * (wy0 * wx0)[:, None]
           + gather(y0, x1) * (wy0 * wx1)[:, None]
           + gather(y1, x0) * (wy1 * wx0)[:, None]
           + gather(y1, x1) * (wy1 * wx1)[:, None])
    return out


def _neighbor_indices(H, W, K):
    nh = (K - 1) // 2
    start_h = jnp.clip(jnp.arange(H) - nh, 0, H - K)
    start_w = jnp.clip(jnp.arange(W) - nh, 0, W - K)
    off = jnp.arange(K)
    nbr_h = start_h[:, None] + off[None, :]
    nbr_w = start_w[:, None] + off[None, :]
    idx = nbr_h[:, None, :, None] * W + nbr_w[None, :, None, :]
    return idx.transpose(2, 3, 0, 1).reshape(K * K, H * W).astype(jnp.int32)


def kernel(wq, bq, wk, bk, wv, bv, wo, bo, off_dw_w, off_dw_b,
           off_ln_g, off_ln_b, off_pw_w, rpe_w, rpe_b, x):
    num_heads, kernel_size = 4, 7
    offset_range_factor = 1.0
    stride = 1
    B, C, H, W = x.shape
    G = num_heads
    gc = C // G
    scale = gc ** (-0.5)
    K = kernel_size
    S = H * W

    x_flat = x.reshape(B, C, S)

    q = conv1x1_pallas(x_flat, wq, bq)
    q_img = q.reshape(B, C, H, W)

    pad = K // 2 if K != stride else 0
    q_off = q_img.reshape(B * G, gc, H, W)
    t = _depthwise_conv(q_off, off_dw_w, off_dw_b, stride=stride, padding=pad)
    t = _layernorm2d(t, off_ln_g, off_ln_b)
    t = jax.nn.gelu(t, approximate=False)
    offset = jnp.einsum("oc,bchw->bohw", off_pw_w, t)
    Hk, Wk = offset.shape[2], offset.shape[3]
    off_range = jnp.array([1.0 / (Hk - 1.0), 1.0 / (Wk - 1.0)],
                          dtype=x.dtype).reshape(1, 2, 1, 1)
    offset = jnp.tanh(offset) * off_range * offset_range_factor
    offset = jnp.transpose(offset, (0, 2, 3, 1))
    ref_pts = _ref_points(Hk, Wk, B * G, x.dtype)
    pos = offset + ref_pts

    grid_xy = pos[..., ::-1]
    x_sampled = _grid_sample_bilinear(x.reshape(B * G, gc, H, W), grid_xy)
    x_sampled = x_sampled.reshape(B, C, S)

    lepe = _depthwise_conv(q_img, rpe_w, rpe_b, stride=1, padding=1)
    lepe_flat = lepe.reshape(B, C, S)

    k = conv1x1_pallas(x_sampled, wk, bk)
    v = conv1x1_pallas(x_sampled, wv, bv)

    idx = _neighbor_indices(H, W, K)
    q_g = q.reshape(B * G, gc, S)
    k_nbr = jnp.transpose(k.reshape(B * G, gc, S)[:, :, idx], (0, 2, 1, 3))
    v_nbr = jnp.transpose(v.reshape(B * G, gc, S)[:, :, idx], (0, 2, 1, 3))

    out = na2d_pallas(q_g, k_nbr, v_nbr, scale=scale)
    out = out.reshape(B, C, S)

    y = conv1x1_pallas(out, wo, bo, residual=lepe_flat)
    return y.reshape(B, C, H, W)


# speedup vs baseline: 4.9068x; 4.9066x over previous
import functools

import jax
import jax.numpy as jnp
from jax import lax
from jax.experimental import pallas as pl
from jax.experimental.pallas import tpu as pltpu

_K = 7
_NH = 3
_GC = 32
_H = 64
_W = 64
_TR = 8


def _conv1x1_kernel(x_ref, w_ref, b_ref, o_ref):
    x = x_ref[0]
    w = w_ref[...]
    y = jnp.dot(w, x, preferred_element_type=jnp.float32)
    o_ref[0] = y + b_ref[...]


def _conv1x1_res_kernel(x_ref, r_ref, w_ref, b_ref, o_ref):
    x = x_ref[0] + r_ref[0]
    w = w_ref[...]
    y = jnp.dot(w, x, preferred_element_type=jnp.float32)
    o_ref[0] = y + b_ref[...]


def _conv1x1(x, w, b, residual=None, *, tile=1024):
    B, C_in, S = x.shape
    C_out = w.shape[0]
    grid = (B, S // tile)

    x_spec = pl.BlockSpec((1, C_in, tile), lambda bi, si: (bi, 0, si))
    w_spec = pl.BlockSpec((C_out, C_in), lambda bi, si: (0, 0))
    b_spec = pl.BlockSpec((C_out, 1), lambda bi, si: (0, 0))
    o_spec = pl.BlockSpec((1, C_out, tile), lambda bi, si: (bi, 0, si))
    b2 = b.reshape(C_out, 1)

    if residual is None:
        kern = _conv1x1_kernel
        operands = (x, w, b2)
        in_specs = [x_spec, w_spec, b_spec]
    else:
        kern = _conv1x1_res_kernel
        operands = (x, residual, w, b2)
        in_specs = [x_spec, x_spec, w_spec, b_spec]

    return pl.pallas_call(
        kern,
        out_shape=jax.ShapeDtypeStruct((B, C_out, S), x.dtype),
        grid=grid,
        in_specs=in_specs,
        out_specs=o_spec,
        compiler_params=pltpu.CompilerParams(
            dimension_semantics=("parallel", "parallel")),
    )(*operands)


def _col_shift(a, dx):
    left = jnp.broadcast_to(a[:, :, dx:dx + 1], (_GC, _H, _NH + 1))
    mid = a[:, :, dx + 1:dx + _W - _K + 1]
    right = jnp.broadcast_to(a[:, :, dx + _W - _K:dx + _W - _K + 1],
                             (_GC, _H, _NH))
    return jnp.concatenate([left, mid, right], axis=2)


def _rows(sref, dx, dy, si):
    r0 = si * _TR
    if si == 0:
        top = jnp.broadcast_to(sref[dx, :, pl.ds(dy, 1), :],
                               (_GC, _NH + 1, _W))
        rest = sref[dx, :, pl.ds(dy + 1, _TR - _NH - 1), :]
        return jnp.concatenate([top, rest], axis=1)
    if si == (_H // _TR) - 1:
        body = sref[dx, :, pl.ds(r0 - _NH + dy, _TR - _NH), :]
        bot = jnp.broadcast_to(sref[dx, :, pl.ds(_H - _K + dy, 1), :],
                               (_GC, _NH, _W))
        return jnp.concatenate([body, bot], axis=1)
    return sref[dx, :, pl.ds(r0 - _NH + dy, _TR), :]


def _na_kernel(q_ref, k_ref, v_ref, o_ref, qs, ks, vs, ls, *, scale):
    qs[...] = q_ref[0, 0] * scale
    k0 = k_ref[0, 0, 0]
    v0 = v_ref[0, 0, 0]
    for dx in range(_K):
        ks[dx] = _col_shift(k0, dx)
        vs[dx] = _col_shift(v0, dx)

    for si in range(_H // _TR):
        r0 = si * _TR
        qsv = qs[:, r0:r0 + _TR, :]

        def pass1(o, m):
            dy = o // _K
            dx = o - dy * _K
            kp = _rows(ks, dx, dy, si)
            lg = jnp.sum(qsv * kp, axis=0)
            ls[o] = lg
            return jnp.maximum(m, lg)

        m = lax.fori_loop(0, _K * _K, pass1,
                          jnp.full((_TR, _W), -jnp.inf, dtype=jnp.float32))

        def pass2(o, carry):
            den, acc = carry
            dy = o // _K
            dx = o - dy * _K
            p = jnp.exp(ls[o] - m)
            vp = _rows(vs, dx, dy, si)
            return den + p, acc + p[None] * vp

        den, acc = lax.fori_loop(
            0, _K * _K, pass2,
            (jnp.zeros((_TR, _W), dtype=jnp.float32),
             jnp.zeros((_GC, _TR, _W), dtype=jnp.float32)))

        inv = pl.reciprocal(den, approx=False)
        o_ref[0, 0, :, r0:r0 + _TR, :] = acc * inv[None]


def _na2d(q, kv, *, scale):
    B, G = q.shape[0], q.shape[1]
    kern = functools.partial(_na_kernel, scale=scale)
    return pl.pallas_call(
        kern,
        out_shape=jax.ShapeDtypeStruct(q.shape, q.dtype),
        grid=(B, G),
        in_specs=[
            pl.BlockSpec((1, 1, _GC, _H, _W), lambda bi, gi: (bi, gi, 0, 0, 0)),
            pl.BlockSpec((1, 1, 1, _GC, _H, _W),
                         lambda bi, gi: (bi, 0, gi, 0, 0, 0)),
            pl.BlockSpec((1, 1, 1, _GC, _H, _W),
                         lambda bi, gi: (bi, 1, gi, 0, 0, 0)),
        ],
        out_specs=pl.BlockSpec((1, 1, _GC, _H, _W),
                               lambda bi, gi: (bi, gi, 0, 0, 0)),
        scratch_shapes=[
            pltpu.VMEM((_GC, _H, _W), jnp.float32),
            pltpu.VMEM((_K, _GC, _H, _W), jnp.float32),
            pltpu.VMEM((_K, _GC, _H, _W), jnp.float32),
            pltpu.VMEM((_K * _K, _TR, _W), jnp.float32),
        ],
        compiler_params=pltpu.CompilerParams(
            dimension_semantics=("parallel", "parallel")),
    )(q, kv, kv)


def _depthwise_conv(x, w, b, *, stride=1, padding=0):
    C = x.shape[1]
    y = lax.conv_general_dilated(
        x, w, window_strides=(stride, stride),
        padding=[(padding, padding), (padding, padding)],
        dimension_numbers=("NCHW", "OIHW", "NCHW"),
        feature_group_count=C)
    if b is not None:
        y = y + b[None, :, None, None]
    return y


def _layernorm2d(x, gamma, beta, eps=1e-6):
    u = jnp.mean(x, axis=1, keepdims=True)
    s = jnp.mean((x - u) ** 2, axis=1, keepdims=True)
    xn = (x - u) / jnp.sqrt(s + eps)
    return gamma[None, :, None, None] * xn + beta[None, :, None, None]


def _ref_points(Hk, Wk, BG, dtype):
    ref_y = (jnp.arange(Hk, dtype=dtype) + 0.5) / (Hk - 1.0) * 2.0 - 1.0
    ref_x = (jnp.arange(Wk, dtype=dtype) + 0.5) / (Wk - 1.0) * 2.0 - 1.0
    ry, rx = jnp.meshgrid(ref_y, ref_x, indexing="ij")
    ref = jnp.stack([ry, rx], axis=-1)
    return jnp.broadcast_to(ref[None], (BG, Hk, Wk, 2))


def _grid_sample_bilinear(img, grid):
    N, C, H, W = img.shape
    gx = (grid[..., 0] + 1.0) * 0.5 * (W - 1)
    gy = (grid[..., 1] + 1.0) * 0.5 * (H - 1)
    x0 = jnp.floor(gx)
    y0 = jnp.floor(gy)
    x1 = x0 + 1.0
    y1 = y0 + 1.0
    wx1 = gx - x0
    wx0 = 1.0 - wx1
    wy1 = gy - y0
    wy0 = 1.0 - wy1
    flat = img.reshape(N, C, H * W)

    def gather(yi, xi):
        valid = (xi >= 0) & (xi <= W - 1) & (yi >= 0) & (yi <= H - 1)
        xc = jnp.clip(xi, 0, W - 1).astype(jnp.int32)
        yc = jnp.clip(yi, 0, H - 1).astype(jnp.int32)
        idx = (yc * W + xc).reshape(N, 1, -1)
        g = jnp.take_along_axis(flat, idx, axis=2).reshape(N, C, *xi.shape[1:])
        return g * valid.astype(img.dtype)[:, None]

    out = (gather(y0, x0) * (wy0 * wx0)[:, None]
           + gather(y0, x1) * (wy0 * wx1)[:, None]
           + gather(y1, x0) * (wy1 * wx0)[:, None]
           + gather(y1, x1) * (wy1 * wx1)[:, None])
    return out


def kernel(wq, bq, wk, bk, wv, bv, wo, bo, off_dw_w, off_dw_b,
           off_ln_g, off_ln_b, off_pw_w, rpe_w, rpe_b, x):
    num_heads = 4
    offset_range_factor = 1.0
    B, C, H, W = x.shape
    G = num_heads
    gc = C // G
    scale = gc ** (-0.5)
    K = _K
    S = H * W

    x_flat = x.reshape(B, C, S)

    q = _conv1x1(x_flat, wq, bq)
    q_img = q.reshape(B, C, H, W)

    q_off = q_img.reshape(B * G, gc, H, W)
    t = _depthwise_conv(q_off, off_dw_w, off_dw_b, stride=1, padding=K // 2)
    t = _layernorm2d(t, off_ln_g, off_ln_b)
    t = jax.nn.gelu(t, approximate=False)
    offset = jnp.einsum("oc,bchw->bohw", off_pw_w, t)
    Hk, Wk = offset.shape[2], offset.shape[3]
    off_range = jnp.array([1.0 / (Hk - 1.0), 1.0 / (Wk - 1.0)],
                          dtype=x.dtype).reshape(1, 2, 1, 1)
    offset = jnp.tanh(offset) * off_range * offset_range_factor
    offset = jnp.transpose(offset, (0, 2, 3, 1))
    pos = offset + _ref_points(Hk, Wk, B * G, x.dtype)

    grid_xy = pos[..., ::-1]
    x_sampled = _grid_sample_bilinear(x.reshape(B * G, gc, H, W), grid_xy)
    x_sampled = x_sampled.reshape(B, C, S)

    lepe = _depthwise_conv(q_img, rpe_w, rpe_b, stride=1, padding=1)
    lepe_flat = lepe.reshape(B, C, S)

    wkv = jnp.concatenate([wk, wv], axis=0)
    bkv = jnp.concatenate([bk, bv], axis=0)
    kv = _conv1x1(x_sampled, wkv, bkv)

    q_g = q.reshape(B, G, gc, H, W)
    kv_g = kv.reshape(B, 2, G, gc, H, W)
    out = _na2d(q_g, kv_g, scale=scale)
    out = out.reshape(B, C, S)

    y = _conv1x1(out, wo, bo, residual=lepe_flat)
    return y.reshape(B, C, H, W)


# speedup vs baseline: 6.2987x; 1.2837x over previous
import functools

import jax
import jax.numpy as jnp
from jax import lax
from jax.experimental import pallas as pl
from jax.experimental.pallas import tpu as pltpu

_K = 7
_NH = 3
_GC = 32
_H = 64
_W = 64
_TR = 8


def _conv1x1_kernel(x_ref, w_ref, b_ref, o_ref):
    x = x_ref[0]
    w = w_ref[...]
    y = jnp.dot(w, x, preferred_element_type=jnp.float32)
    o_ref[0] = y + b_ref[...]


def _conv1x1_res_kernel(x_ref, r_ref, w_ref, b_ref, o_ref):
    x = x_ref[0] + r_ref[0]
    w = w_ref[...]
    y = jnp.dot(w, x, preferred_element_type=jnp.float32)
    o_ref[0] = y + b_ref[...]


def _conv1x1(x, w, b, residual=None, *, tile=1024):
    B, C_in, S = x.shape
    C_out = w.shape[0]
    grid = (B, S // tile)

    x_spec = pl.BlockSpec((1, C_in, tile), lambda bi, si: (bi, 0, si))
    w_spec = pl.BlockSpec((C_out, C_in), lambda bi, si: (0, 0))
    b_spec = pl.BlockSpec((C_out, 1), lambda bi, si: (0, 0))
    o_spec = pl.BlockSpec((1, C_out, tile), lambda bi, si: (bi, 0, si))
    b2 = b.reshape(C_out, 1)

    if residual is None:
        kern = _conv1x1_kernel
        operands = (x, w, b2)
        in_specs = [x_spec, w_spec, b_spec]
    else:
        kern = _conv1x1_res_kernel
        operands = (x, residual, w, b2)
        in_specs = [x_spec, x_spec, w_spec, b_spec]

    return pl.pallas_call(
        kern,
        out_shape=jax.ShapeDtypeStruct((B, C_out, S), x.dtype),
        grid=grid,
        in_specs=in_specs,
        out_specs=o_spec,
        compiler_params=pltpu.CompilerParams(
            dimension_semantics=("parallel", "parallel")),
    )(*operands)


def _col_shift(a, dx):
    left = jnp.broadcast_to(a[:, :, dx:dx + 1], (_GC, _H, _NH + 1))
    mid = a[:, :, dx + 1:dx + _W - _K + 1]
    right = jnp.broadcast_to(a[:, :, dx + _W - _K:dx + _W - _K + 1],
                             (_GC, _H, _NH))
    return jnp.concatenate([left, mid, right], axis=2)


def _rows(sref, dx, dy, si):
    r0 = si * _TR
    if si == 0:
        top = jnp.broadcast_to(sref[dx, :, pl.ds(dy, 1), :],
                               (_GC, _NH + 1, _W))
        rest = sref[dx, :, pl.ds(dy + 1, _TR - _NH - 1), :]
        return jnp.concatenate([top, rest], axis=1)
    if si == (_H // _TR) - 1:
        body = sref[dx, :, pl.ds(r0 - _NH + dy, _TR - _NH), :]
        bot = jnp.broadcast_to(sref[dx, :, pl.ds(_H - _K + dy, 1), :],
                               (_GC, _NH, _W))
        return jnp.concatenate([body, bot], axis=1)
    return sref[dx, :, pl.ds(r0 - _NH + dy, _TR), :]


def _na_kernel(q_ref, k_ref, v_ref, o_ref, qs, ks, vs, ls, *, scale):
    qs[...] = q_ref[0, 0] * scale
    k0 = k_ref[0, 0, 0]
    v0 = v_ref[0, 0, 0]
    for dx in range(_K):
        ks[dx] = _col_shift(k0, dx)
        vs[dx] = _col_shift(v0, dx)

    for si in range(_H // _TR):
        r0 = si * _TR
        qsv = qs[:, r0:r0 + _TR, :]

        def pass1(o, m):
            dy = o // _K
            dx = o - dy * _K
            kp = _rows(ks, dx, dy, si)
            lg = jnp.sum(qsv * kp, axis=0)
            ls[o] = lg
            return jnp.maximum(m, lg)

        m = lax.fori_loop(0, _K * _K, pass1,
                          jnp.full((_TR, _W), -jnp.inf, dtype=jnp.float32))

        def pass2(o, carry):
            den, acc = carry
            dy = o // _K
            dx = o - dy * _K
            p = jnp.exp(ls[o] - m)
            vp = _rows(vs, dx, dy, si)
            return den + p, acc + p[None] * vp

        den, acc = lax.fori_loop(
            0, _K * _K, pass2,
            (jnp.zeros((_TR, _W), dtype=jnp.float32),
             jnp.zeros((_GC, _TR, _W), dtype=jnp.float32)))

        inv = pl.reciprocal(den, approx=False)
        o_ref[0, 0, :, r0:r0 + _TR, :] = acc * inv[None]


def _na2d(q, kv, *, scale):
    B, G = q.shape[0], q.shape[1]
    kern = functools.partial(_na_kernel, scale=scale)
    return pl.pallas_call(
        kern,
        out_shape=jax.ShapeDtypeStruct(q.shape, q.dtype),
        grid=(B, G),
        in_specs=[
            pl.BlockSpec((1, 1, _GC, _H, _W), lambda bi, gi: (bi, gi, 0, 0, 0)),
            pl.BlockSpec((1, 1, 1, _GC, _H, _W),
                         lambda bi, gi: (bi, 0, gi, 0, 0, 0)),
            pl.BlockSpec((1, 1, 1, _GC, _H, _W),
                         lambda bi, gi: (bi, 1, gi, 0, 0, 0)),
        ],
        out_specs=pl.BlockSpec((1, 1, _GC, _H, _W),
                               lambda bi, gi: (bi, gi, 0, 0, 0)),
        scratch_shapes=[
            pltpu.VMEM((_GC, _H, _W), jnp.float32),
            pltpu.VMEM((_K, _GC, _H, _W), jnp.float32),
            pltpu.VMEM((_K, _GC, _H, _W), jnp.float32),
            pltpu.VMEM((_K * _K, _TR, _W), jnp.float32),
        ],
        compiler_params=pltpu.CompilerParams(
            dimension_semantics=("parallel", "parallel")),
    )(q, kv, kv)


def _sample_kernel(x_ref, off_ref, o_ref):
    H, W = _H, _W
    o = off_ref[0, 0]
    offy = jnp.tanh(o[0]) * jnp.float32(1.0 / (H - 1))
    offx = jnp.tanh(o[1]) * jnp.float32(1.0 / (W - 1))
    iy = jax.lax.broadcasted_iota(jnp.int32, (H, W), 0).astype(jnp.float32)
    ix = jax.lax.broadcasted_iota(jnp.int32, (H, W), 1).astype(jnp.float32)
    ref_y = (iy + 0.5) / (H - 1.0) * 2.0 - 1.0
    ref_x = (ix + 0.5) / (W - 1.0) * 2.0 - 1.0
    gy = (offy + ref_y + 1.0) * 0.5 * (H - 1)
    gx = (offx + ref_x + 1.0) * 0.5 * (W - 1)
    wy1 = gy - iy
    wy0 = 1.0 - wy1
    wx1 = gx - ix
    wx0 = 1.0 - wx1

    xx = x_ref[0, 0]
    zc = jnp.zeros((_GC, H, 1), dtype=jnp.float32)
    zr = jnp.zeros((_GC, 1, W), dtype=jnp.float32)
    x_e = jnp.concatenate([xx[:, :, 1:], zc], axis=2)
    x_s = jnp.concatenate([xx[:, 1:, :], zr], axis=1)
    x_se = jnp.concatenate([x_e[:, 1:, :], zr], axis=1)

    out = (xx * (wy0 * wx0)[None] + x_e * (wy0 * wx1)[None]
           + x_s * (wy1 * wx0)[None] + x_se * (wy1 * wx1)[None])
    o_ref[0, 0] = out


def _deform_sample(x_g, off_raw):
    B, G = x_g.shape[0], x_g.shape[1]
    return pl.pallas_call(
        _sample_kernel,
        out_shape=jax.ShapeDtypeStruct(x_g.shape, x_g.dtype),
        grid=(B, G),
        in_specs=[
            pl.BlockSpec((1, 1, _GC, _H, _W), lambda bi, gi: (bi, gi, 0, 0, 0)),
            pl.BlockSpec((1, 1, 2, _H, _W), lambda bi, gi: (bi, gi, 0, 0, 0)),
        ],
        out_specs=pl.BlockSpec((1, 1, _GC, _H, _W),
                               lambda bi, gi: (bi, gi, 0, 0, 0)),
        compiler_params=pltpu.CompilerParams(
            dimension_semantics=("parallel", "parallel")),
    )(x_g, off_raw)


def _depthwise_conv(x, w, b, *, stride=1, padding=0):
    C = x.shape[1]
    y = lax.conv_general_dilated(
        x, w, window_strides=(stride, stride),
        padding=[(padding, padding), (padding, padding)],
        dimension_numbers=("NCHW", "OIHW", "NCHW"),
        feature_group_count=C)
    if b is not None:
        y = y + b[None, :, None, None]
    return y


def _layernorm2d(x, gamma, beta, eps=1e-6):
    u = jnp.mean(x, axis=1, keepdims=True)
    s = jnp.mean((x - u) ** 2, axis=1, keepdims=True)
    xn = (x - u) / jnp.sqrt(s + eps)
    return gamma[None, :, None, None] * xn + beta[None, :, None, None]


def kernel(wq, bq, wk, bk, wv, bv, wo, bo, off_dw_w, off_dw_b,
           off_ln_g, off_ln_b, off_pw_w, rpe_w, rpe_b, x):
    num_heads = 4
    offset_range_factor = 1.0
    B, C, H, W = x.shape
    G = num_heads
    gc = C // G
    scale = gc ** (-0.5)
    K = _K
    S = H * W

    x_flat = x.reshape(B, C, S)

    q = _conv1x1(x_flat, wq, bq)
    q_img = q.reshape(B, C, H, W)

    q_off = q_img.reshape(B * G, gc, H, W)
    t = _depthwise_conv(q_off, off_dw_w, off_dw_b, stride=1, padding=K // 2)
    t = _layernorm2d(t, off_ln_g, off_ln_b)
    t = jax.nn.gelu(t, approximate=False)
    off_raw = jnp.einsum("oc,bchw->bohw", off_pw_w, t)

    x_sampled = _deform_sample(x.reshape(B, G, gc, H, W),
                               off_raw.reshape(B, G, 2, H, W))
    x_sampled = x_sampled.reshape(B, C, S)

    lepe = _depthwise_conv(q_img, rpe_w, rpe_b, stride=1, padding=1)
    lepe_flat = lepe.reshape(B, C, S)

    wkv = jnp.concatenate([wk, wv], axis=0)
    bkv = jnp.concatenate([bk, bv], axis=0)
    kv = _conv1x1(x_sampled, wkv, bkv)

    q_g = q.reshape(B, G, gc, H, W)
    kv_g = kv.reshape(B, 2, G, gc, H, W)
    out = _na2d(q_g, kv_g, scale=scale)
    out = out.reshape(B, C, S)

    y = _conv1x1(out, wo, bo, residual=lepe_flat)
    return y.reshape(B, C, H, W)


# speedup vs baseline: 8.2959x; 1.3171x over previous
import functools

import jax
import jax.numpy as jnp
from jax import lax
from jax.experimental import pallas as pl
from jax.experimental.pallas import tpu as pltpu

_K = 7
_NH = 3
_GC = 32
_H = 64
_W = 64
_TR = 8


def _conv1x1_kernel(x_ref, w_ref, b_ref, o_ref):
    x = x_ref[0]
    w = w_ref[...]
    y = jnp.dot(w, x, preferred_element_type=jnp.float32)
    o_ref[0] = y + b_ref[...]


def _conv1x1_res_kernel(x_ref, r_ref, w_ref, b_ref, o_ref):
    x = x_ref[0] + r_ref[0]
    w = w_ref[...]
    y = jnp.dot(w, x, preferred_element_type=jnp.float32)
    o_ref[0] = y + b_ref[...]


def _conv1x1(x, w, b, residual=None, *, tile=1024):
    B, C_in, S = x.shape
    C_out = w.shape[0]
    grid = (B, S // tile)

    x_spec = pl.BlockSpec((1, C_in, tile), lambda bi, si: (bi, 0, si))
    w_spec = pl.BlockSpec((C_out, C_in), lambda bi, si: (0, 0))
    b_spec = pl.BlockSpec((C_out, 1), lambda bi, si: (0, 0))
    o_spec = pl.BlockSpec((1, C_out, tile), lambda bi, si: (bi, 0, si))
    b2 = b.reshape(C_out, 1)

    if residual is None:
        kern = _conv1x1_kernel
        operands = (x, w, b2)
        in_specs = [x_spec, w_spec, b_spec]
    else:
        kern = _conv1x1_res_kernel
        operands = (x, residual, w, b2)
        in_specs = [x_spec, x_spec, w_spec, b_spec]

    return pl.pallas_call(
        kern,
        out_shape=jax.ShapeDtypeStruct((B, C_out, S), x.dtype),
        grid=grid,
        in_specs=in_specs,
        out_specs=o_spec,
        compiler_params=pltpu.CompilerParams(
            dimension_semantics=("parallel", "parallel")),
    )(*operands)


_W2 = 2 * _W


def _cshift_c(sref, dx, a, b):
    n = b - a
    pieces = []
    for h in range(2):
        base = h * _W
        left = jnp.broadcast_to(sref[a:b, :, base + dx:base + dx + 1],
                                (n, _H, _NH + 1))
        mid = sref[a:b, :, base + dx + 1:base + dx + _W - _K + 1]
        right = jnp.broadcast_to(
            sref[a:b, :, base + dx + _W - _K:base + dx + _W - _K + 1],
            (n, _H, _NH))
        pieces += [left, mid, right]
    return jnp.concatenate(pieces, axis=2)


def _shift_rows2d(a, sh, rows, w):
    if sh == 0:
        return a
    if sh > 0:
        body = a[..., sh:, :]
        tail = jnp.broadcast_to(a[..., rows - 1:rows, :], a.shape[:-2] + (sh, w))
        return jnp.concatenate([body, tail], axis=-2)
    head = jnp.broadcast_to(a[..., 0:1, :], a.shape[:-2] + (-sh, w))
    body = a[..., :rows + sh, :]
    return jnp.concatenate([head, body], axis=-2)


def _rows_p(sref, dx, dy, si):
    r0 = si * _TR
    if si == 0:
        top = jnp.broadcast_to(sref[dx, :, pl.ds(dy, 1), :],
                               (_GC, _NH + 1, _W2))
        rest = sref[dx, :, pl.ds(dy + 1, _TR - _NH - 1), :]
        return jnp.concatenate([top, rest], axis=1)
    body = sref[dx, :, pl.ds(r0 - _NH + dy, _TR - _NH), :]
    bot = jnp.broadcast_to(sref[dx, :, pl.ds(_H - _K + dy, 1), :],
                           (_GC, _NH, _W2))
    return jnp.concatenate([body, bot], axis=1)


def _na_kernel(q_ref, k_ref, v_ref, o_ref, qs3, kvs, ls, asc, oacc, *, scale):
    KK = _K * _K
    CH = 8
    chunks = [(c, c + CH) for c in range(0, _GC, CH)]
    SI_LAST = (_H // _TR) - 1

    for a, b in chunks:
        qs3[a:b] = jnp.concatenate(
            [q_ref[0, 0, a:b], q_ref[0, 1, a:b]], axis=2) * scale

    for a, b in chunks:
        asc[a:b] = jnp.concatenate(
            [k_ref[0, 0, 0, a:b], k_ref[0, 0, 1, a:b]], axis=2)
    for dx in range(_K):
        for a, b in chunks:
            kvs[dx, a:b] = _cshift_c(asc, dx, a, b)

    m = jnp.full((_H, _W2), -jnp.inf, dtype=jnp.float32)
    for dy in range(_K):
        sh = _NH - dy

        def p1(dx, m, dy=dy, sh=sh):
            lu = jnp.zeros((_H, _W2), dtype=jnp.float32)
            for a, b in chunks:
                qt = _shift_rows2d(qs3[a:b], sh, _H, _W2)
                lu = lu + jnp.sum(qt * kvs[dx, a:b], axis=0)
            lg = _shift_rows2d(lu, -sh, _H, _W2)
            ls[dy * _K + dx] = lg
            return jnp.maximum(m, lg)

        m = lax.fori_loop(0, _K, p1, m)

    me_all = []
    for si in (0, SI_LAST):
        r0 = si * _TR
        qsv = qs3[:, r0:r0 + _TR, :]

        def e1(o, me, si=si, qsv=qsv, r0=r0):
            dy = o // _K
            dx = o - dy * _K
            kp = _rows_p(kvs, dx, dy, si)
            lg = jnp.sum(qsv * kp, axis=0)
            ls[o, r0:r0 + _TR] = lg
            return jnp.maximum(me, lg)

        me_all.append(lax.fori_loop(
            0, KK, e1, jnp.full((_TR, _W2), -jnp.inf, dtype=jnp.float32)))

    for a, b in chunks:
        asc[a:b] = jnp.concatenate(
            [v_ref[0, 0, 0, a:b], v_ref[0, 0, 1, a:b]], axis=2)
    for dx in range(_K):
        for a, b in chunks:
            kvs[dx, a:b] = _cshift_c(asc, dx, a, b)

    den = jnp.zeros((_H, _W2), dtype=jnp.float32)
    for a, b in chunks:
        oacc[a:b] = jnp.zeros((CH, _H, _W2), dtype=jnp.float32)
    for dy in range(_K):
        sh = _NH - dy
        for a, b in chunks:
            asc[a:b] = jnp.zeros((CH, _H, _W2), dtype=jnp.float32)

        def p2(dx, den, dy=dy, sh=sh):
            p = jnp.exp(ls[dy * _K + dx] - m)
            pt = _shift_rows2d(p, sh, _H, _W2)
            for a, b in chunks:
                asc[a:b] = asc[a:b] + pt[None] * kvs[dx, a:b]
            return den + p

        den = lax.fori_loop(0, _K, p2, den)
        for a, b in chunks:
            oacc[a:b] = oacc[a:b] + _shift_rows2d(asc[a:b], -sh, _H, _W2)

    inv = pl.reciprocal(den, approx=False)
    for a, b in chunks:
        res = oacc[a:b] * inv[None]
        o_ref[0, 0, a:b] = res[:, :, :_W]
        o_ref[0, 1, a:b] = res[:, :, _W:]

    for si, me in zip((0, SI_LAST), me_all):
        r0 = si * _TR

        def e2(o, carry, si=si, r0=r0, me=me):
            dene, acce = carry
            dy = o // _K
            dx = o - dy * _K
            p = jnp.exp(ls[o, r0:r0 + _TR] - me)
            vp = _rows_p(kvs, dx, dy, si)
            return dene + p, acce + p[None] * vp

        dene, acce = lax.fori_loop(
            0, KK, e2,
            (jnp.zeros((_TR, _W2), dtype=jnp.float32),
             jnp.zeros((_GC, _TR, _W2), dtype=jnp.float32)))

        inve = pl.reciprocal(dene, approx=False)
        rese = acce * inve[None]
        o_ref[0, 0, :, r0:r0 + _TR, :] = rese[:, :, :_W]
        o_ref[0, 1, :, r0:r0 + _TR, :] = rese[:, :, _W:]


def _na2d(q, kv, *, scale):
    B, G = q.shape[0], q.shape[1]
    kern = functools.partial(_na_kernel, scale=scale)
    return pl.pallas_call(
        kern,
        out_shape=jax.ShapeDtypeStruct(q.shape, q.dtype),
        grid=(B, G // 2),
        in_specs=[
            pl.BlockSpec((1, 2, _GC, _H, _W), lambda bi, pi: (bi, pi, 0, 0, 0)),
            pl.BlockSpec((1, 1, 2, _GC, _H, _W),
                         lambda bi, pi: (bi, 0, pi, 0, 0, 0)),
            pl.BlockSpec((1, 1, 2, _GC, _H, _W),
                         lambda bi, pi: (bi, 1, pi, 0, 0, 0)),
        ],
        out_specs=pl.BlockSpec((1, 2, _GC, _H, _W),
                               lambda bi, pi: (bi, pi, 0, 0, 0)),
        scratch_shapes=[
            pltpu.VMEM((_GC, _H, _W2), jnp.float32),
            pltpu.VMEM((_K, _GC, _H, _W2), jnp.float32),
            pltpu.VMEM((_K * _K, _H, _W2), jnp.float32),
            pltpu.VMEM((_GC, _H, _W2), jnp.float32),
            pltpu.VMEM((_GC, _H, _W2), jnp.float32),
        ],
        compiler_params=pltpu.CompilerParams(
            dimension_semantics=("parallel", "parallel")),
    )(q, kv, kv)


def _sample_kernel(x_ref, off_ref, o_ref):
    H, W = _H, _W
    o = off_ref[0, 0]
    offy = jnp.tanh(o[0]) * jnp.float32(1.0 / (H - 1))
    offx = jnp.tanh(o[1]) * jnp.float32(1.0 / (W - 1))
    iy = jax.lax.broadcasted_iota(jnp.int32, (H, W), 0).astype(jnp.float32)
    ix = jax.lax.broadcasted_iota(jnp.int32, (H, W), 1).astype(jnp.float32)
    ref_y = (iy + 0.5) / (H - 1.0) * 2.0 - 1.0
    ref_x = (ix + 0.5) / (W - 1.0) * 2.0 - 1.0
    gy = (offy + ref_y + 1.0) * 0.5 * (H - 1)
    gx = (offx + ref_x + 1.0) * 0.5 * (W - 1)
    wy1 = gy - iy
    wy0 = 1.0 - wy1
    wx1 = gx - ix
    wx0 = 1.0 - wx1

    xx = x_ref[0, 0]
    zc = jnp.zeros((_GC, H, 1), dtype=jnp.float32)
    zr = jnp.zeros((_GC, 1, W), dtype=jnp.float32)
    x_e = jnp.concatenate([xx[:, :, 1:], zc], axis=2)
    x_s = jnp.concatenate([xx[:, 1:, :], zr], axis=1)
    x_se = jnp.concatenate([x_e[:, 1:, :], zr], axis=1)

    out = (xx * (wy0 * wx0)[None] + x_e * (wy0 * wx1)[None]
           + x_s * (wy1 * wx0)[None] + x_se * (wy1 * wx1)[None])
    o_ref[0, 0] = out


def _deform_sample(x_g, off_raw):
    B, G = x_g.shape[0], x_g.shape[1]
    return pl.pallas_call(
        _sample_kernel,
        out_shape=jax.ShapeDtypeStruct(x_g.shape, x_g.dtype),
        grid=(B, G),
        in_specs=[
            pl.BlockSpec((1, 1, _GC, _H, _W), lambda bi, gi: (bi, gi, 0, 0, 0)),
            pl.BlockSpec((1, 1, 2, _H, _W), lambda bi, gi: (bi, gi, 0, 0, 0)),
        ],
        out_specs=pl.BlockSpec((1, 1, _GC, _H, _W),
                               lambda bi, gi: (bi, gi, 0, 0, 0)),
        compiler_params=pltpu.CompilerParams(
            dimension_semantics=("parallel", "parallel")),
    )(x_g, off_raw)


def _depthwise_conv(x, w, b, *, stride=1, padding=0):
    C = x.shape[1]
    y = lax.conv_general_dilated(
        x, w, window_strides=(stride, stride),
        padding=[(padding, padding), (padding, padding)],
        dimension_numbers=("NCHW", "OIHW", "NCHW"),
        feature_group_count=C)
    if b is not None:
        y = y + b[None, :, None, None]
    return y


def _layernorm2d(x, gamma, beta, eps=1e-6):
    u = jnp.mean(x, axis=1, keepdims=True)
    s = jnp.mean((x - u) ** 2, axis=1, keepdims=True)
    xn = (x - u) / jnp.sqrt(s + eps)
    return gamma[None, :, None, None] * xn + beta[None, :, None, None]


def kernel(wq, bq, wk, bk, wv, bv, wo, bo, off_dw_w, off_dw_b,
           off_ln_g, off_ln_b, off_pw_w, rpe_w, rpe_b, x):
    num_heads = 4
    offset_range_factor = 1.0
    B, C, H, W = x.shape
    G = num_heads
    gc = C // G
    scale = gc ** (-0.5)
    K = _K
    S = H * W

    x_flat = x.reshape(B, C, S)

    q = _conv1x1(x_flat, wq, bq)
    q_img = q.reshape(B, C, H, W)

    q_off = q_img.reshape(B * G, gc, H, W)
    t = _depthwise_conv(q_off, off_dw_w, off_dw_b, stride=1, padding=K // 2)
    t = _layernorm2d(t, off_ln_g, off_ln_b)
    t = jax.nn.gelu(t, approximate=False)
    off_raw = jnp.einsum("oc,bchw->bohw", off_pw_w, t)

    x_sampled = _deform_sample(x.reshape(B, G, gc, H, W),
                               off_raw.reshape(B, G, 2, H, W))
    x_sampled = x_sampled.reshape(B, C, S)

    lepe = _depthwise_conv(q_img, rpe_w, rpe_b, stride=1, padding=1)
    lepe_flat = lepe.reshape(B, C, S)

    wkv = jnp.concatenate([wk, wv], axis=0)
    bkv = jnp.concatenate([bk, bv], axis=0)
    kv = _conv1x1(x_sampled, wkv, bkv)

    q_g = q.reshape(B, G, gc, H, W)
    kv_g = kv.reshape(B, 2, G, gc, H, W)
    out = _na2d(q_g, kv_g, scale=scale)
    out = out.reshape(B, C, S)

    y = _conv1x1(out, wo, bo, residual=lepe_flat)
    return y.reshape(B, C, H, W)
